# Initial kernel scaffold; baseline (speedup 1.0000x reference)
import jax
import jax.numpy as jnp
from jax.experimental import pallas as pl


def kernel(x, edge_index, batch, W1, b1, W2, b2, Wc, bc):
    raise NotImplementedError("write your pallas kernel here")


# speedup vs baseline: 11.3207x; 11.3207x over previous
import functools

import jax
import jax.numpy as jnp
from jax import lax
from jax.experimental import pallas as pl
from jax.experimental.pallas import tpu as pltpu
from jax.experimental.pallas import tpu_sc as plsc

f32 = jnp.float32
i32 = jnp.int32

N = 10000
E = 320000
D = 128
G = 64
C = 32

NW = 32
CH = 128
NCH = -(-E // (NW * CH))
EPAD = NW * NCH * CH
CE = NCH * CH

NPAD = 10240
RPT = NPAD // 16
BLK = 512
NBLK = NPAD // BLK

def _sc_mesh():
    return plsc.VectorSubcoreMesh(core_axis_name="c", subcore_axis_name="s",
                                  num_cores=2, num_subcores=16)


def _deg_body(dst_hbm, out_hbm, acc_v, idx_v):
    cid = lax.axis_index("c")
    sid = lax.axis_index("s")
    wid = cid * 16 + sid

    def zero(i, carry):
        acc_v[pl.ds(i * 16, 16)] = jnp.zeros((16,), f32)
        return carry

    lax.fori_loop(0, NPAD // 16, zero, 0)

    ones = jnp.ones((16,), f32)
    base = wid * CE

    def chunk(k, carry):
        pltpu.sync_copy(dst_hbm.at[pl.ds(base + k * CH, CH)], idx_v)

        def inner(j, c2):
            idx = idx_v[pl.ds(j * 16, 16)]
            plsc.addupdate_scatter(acc_v, [idx], ones)
            return c2

        lax.fori_loop(0, CH // 16, inner, 0)
        return carry

    lax.fori_loop(0, NCH, chunk, 0)
    pltpu.sync_copy(acc_v, out_hbm.at[wid])


def _deg_call(dst_p):
    return pl.kernel(
        _deg_body,
        out_type=jax.ShapeDtypeStruct((NW, NPAD), f32),
        mesh=_sc_mesh(),
        scratch_types=[
            pltpu.VMEM((NPAD,), f32),
            pltpu.VMEM((CH,), i32),
        ],
        compiler_params=pltpu.CompilerParams(needs_layout_passes=False),
    )(dst_p)


def _scatter_body(y_hbm, src_hbm, dst_hbm, zeros_hbm, out_hbm,
                  sidx, didx, rows, shared, gsem):
    cid = lax.axis_index("c")
    sid = lax.axis_index("s")
    wid = cid * 16 + sid
    r0 = sid * RPT

    pltpu.sync_copy(zeros_hbm.at[pl.ds(r0, RPT)], shared.at[pl.ds(r0, RPT)])
    plsc.subcore_barrier()

    base = wid * CE

    def chunk(k, carry):
        pltpu.sync_copy(src_hbm.at[pl.ds(base + k * CH, CH)], sidx)
        gather = pltpu.async_copy(y_hbm.at[sidx], rows, gsem)
        pltpu.sync_copy(dst_hbm.at[pl.ds(base + k * CH, CH)], didx)
        gather.wait()
        pltpu.sync_copy(rows, shared.at[didx], add=True)
        return carry

    lax.fori_loop(0, NCH, chunk, 0)
    plsc.subcore_barrier()
    pltpu.sync_copy(shared.at[pl.ds(r0, RPT)], out_hbm.at[cid, pl.ds(r0, RPT)])


def _scatter_call(y, src_p, dst_p, zeros_big):
    return pl.kernel(
        _scatter_body,
        out_type=jax.ShapeDtypeStruct((2, NPAD, D), f32),
        mesh=_sc_mesh(),
        scratch_types=[
            pltpu.VMEM((CH,), i32),
            pltpu.VMEM((CH,), i32),
            pltpu.VMEM((CH, D), f32),
            pltpu.VMEM_SHARED((NPAD, D), f32),
            pltpu.SemaphoreType.DMA,
        ],
        compiler_params=pltpu.CompilerParams(needs_layout_passes=False),
    )(y, src_p, dst_p, zeros_big)


def _dinv_of(deg_ref):
    return lax.rsqrt(jnp.sum(deg_ref[...], axis=0) + 1.0)[:, None]


def _k1_body(x_ref, w_ref, deg_ref, y_ref):
    xw = jnp.dot(x_ref[...], w_ref[...], preferred_element_type=f32)
    y_ref[...] = xw * _dinv_of(deg_ref)


def _k1(x_p, W1, deg_parts):
    return pl.pallas_call(
        _k1_body,
        grid=(NBLK,),
        in_specs=[
            pl.BlockSpec((BLK, D), lambda i: (i, 0)),
            pl.BlockSpec((D, D), lambda i: (0, 0)),
            pl.BlockSpec((NW, BLK), lambda i: (0, i)),
        ],
        out_specs=pl.BlockSpec((BLK, D), lambda i: (i, 0)),
        out_shape=jax.ShapeDtypeStruct((NPAD, D), f32),
    )(x_p, W1, deg_parts)


def _kmid_body(sp_ref, y1_ref, deg_ref, b_ref, w_ref, y2_ref):
    dinv = _dinv_of(deg_ref)
    s = sp_ref[0] + sp_ref[1] + y1_ref[...]
    h = jnp.maximum(s * dinv + b_ref[...], 0.0)
    y2_ref[...] = jnp.dot(h, w_ref[...], preferred_element_type=f32) * dinv


def _kmid(s1, y1, deg_parts, b1, W2):
    return pl.pallas_call(
        _kmid_body,
        grid=(NBLK,),
        in_specs=[
            pl.BlockSpec((2, BLK, D), lambda i: (0, i, 0)),
            pl.BlockSpec((BLK, D), lambda i: (i, 0)),
            pl.BlockSpec((NW, BLK), lambda i: (0, i)),
            pl.BlockSpec((1, D), lambda i: (0, 0)),
            pl.BlockSpec((D, D), lambda i: (0, 0)),
        ],
        out_specs=pl.BlockSpec((BLK, D), lambda i: (i, 0)),
        out_shape=jax.ShapeDtypeStruct((NPAD, D), f32),
    )(s1, y1, deg_parts, b1, W2)


def _kfin_body(sp_ref, y2_ref, deg_ref, b_ref, batch_ref, wc_ref, bc_ref,
               out_ref, acc, cnt):
    i = pl.program_id(0)

    @pl.when(i == 0)
    def _init():
        acc[...] = jnp.zeros_like(acc)
        cnt[...] = jnp.zeros_like(cnt)

    dinv = _dinv_of(deg_ref)
    h = jnp.maximum((sp_ref[0] + sp_ref[1] + y2_ref[...]) * dinv + b_ref[...],
                    0.0)
    row = i * BLK + lax.broadcasted_iota(i32, (1, BLK), 1)
    seg = lax.broadcasted_iota(i32, (G, BLK), 0)
    onehot = jnp.where((seg == batch_ref[0]) & (row < N), 1.0, 0.0)
    acc[...] += jnp.dot(onehot, h, preferred_element_type=f32)
    cnt[...] = cnt[...] + jnp.sum(onehot, axis=1, keepdims=True)

    @pl.when(i == NBLK - 1)
    def _fin():
        pooled = acc[...] / jnp.maximum(cnt[...], 1.0)
        out_ref[...] = lax.dot_general(
            pooled, wc_ref[...], (((1,), (1,)), ((), ())),
            preferred_element_type=f32) + bc_ref[...]


def _kfin(s2, y2, deg_parts, b2, batch3d, Wc, bc):
    return pl.pallas_call(
        _kfin_body,
        grid=(NBLK,),
        in_specs=[
            pl.BlockSpec((2, BLK, D), lambda i: (0, i, 0)),
            pl.BlockSpec((BLK, D), lambda i: (i, 0)),
            pl.BlockSpec((NW, BLK), lambda i: (0, i)),
            pl.BlockSpec((1, D), lambda i: (0, 0)),
            pl.BlockSpec((1, 1, BLK), lambda i: (i, 0, 0)),
            pl.BlockSpec((C, D), lambda i: (0, 0)),
            pl.BlockSpec((1, C), lambda i: (0, 0)),
        ],
        out_specs=pl.BlockSpec((G, C), lambda i: (0, 0)),
        out_shape=jax.ShapeDtypeStruct((G, C), f32),
        scratch_shapes=[
            pltpu.VMEM((G, D), f32),
            pltpu.VMEM((G, D), f32),
        ],
    )(s2, y2, deg_parts, b2, batch3d, Wc, bc)


def kernel(x, edge_index, batch, W1, b1, W2, b2, Wc, bc):
    epad = jnp.full((EPAD - E,), N, i32)
    src_p = jnp.concatenate([edge_index[0], epad])
    dst_p = jnp.concatenate([edge_index[1], epad])
    x_p = jnp.pad(x, ((0, NPAD - N), (0, 0)))
    batch3d = jnp.pad(batch, (0, NPAD - N)).reshape(NBLK, 1, BLK)
    zeros_big = jnp.zeros((NPAD, D), f32)

    deg_parts = _deg_call(dst_p)
    y1 = _k1(x_p, W1, deg_parts)
    s1 = _scatter_call(y1, src_p, dst_p, zeros_big)
    y2 = _kmid(s1, y1, deg_parts, b1.reshape(1, D), W2)
    s2 = _scatter_call(y2, src_p, dst_p, zeros_big)
    return _kfin(s2, y2, deg_parts, b2.reshape(1, D), batch3d, Wc,
                 bc.reshape(1, C))
